# initial kernel scaffold (unmeasured)
import jax
import jax.numpy as jnp
from jax import lax
from jax.experimental import pallas as pl
from jax.experimental.pallas import tpu as pltpu

N_DEV = 4
SQ = 1024
SKV = 1024
HQ_PER = 8
DH = 128
DM = 1024
BLK = 64
SCALE = 0.08838834764831843


def _body(x_ref, wq_ref, k_ref, v_ref, wo_ref, out_ref,
          comm_ref, send_sems, recv_sems):
    my = lax.axis_index("i")
    left = lax.rem(my + N_DEV - 1, N_DEV)
    right = lax.rem(my + 1, N_DEV)

    barrier = pltpu.get_barrier_semaphore()
    pl.semaphore_signal(barrier, inc=1, device_id=(left,),
                        device_id_type=pl.DeviceIdType.MESH)
    pl.semaphore_signal(barrier, inc=1, device_id=(right,),
                        device_id_type=pl.DeviceIdType.MESH)
    pl.semaphore_wait(barrier, 2)

    comm_ref[0, 0] = wq_ref[...]
    comm_ref[0, 1] = wo_ref[...]

    for h in range(N_DEV - 1):
        rdma = pltpu.make_async_remote_copy(
            src_ref=comm_ref.at[h],
            dst_ref=comm_ref.at[h + 1],
            send_sem=send_sems.at[h],
            recv_sem=recv_sems.at[h + 1],
            device_id=(right,),
            device_id_type=pl.DeviceIdType.MESH,
        )
        rdma.start()
        rdma.wait()

    row = lax.broadcasted_iota(jnp.int32, (SQ, 1), 0) + my * SQ
    col = lax.broadcasted_iota(jnp.int32, (1, SKV), 1)
    qb = row // BLK
    kb = col // BLK
    keep = (qb == kb) | (kb == 0) | (lax.rem(qb + kb, 3) == 0)
    addmask = jnp.where(keep, 0.0, -1e9).astype(jnp.float32)

    xb = x_ref[...]
    for s in range(N_DEV):
        j = lax.rem(my + N_DEV - s, N_DEV)
        q = jnp.dot(xb, comm_ref[s, 0],
                    preferred_element_type=jnp.float32).astype(jnp.bfloat16)
        kblk = k_ref[pl.ds(j * HQ_PER, HQ_PER)]
        vblk = v_ref[pl.ds(j * HQ_PER, HQ_PER)]
        ctxs = []
        for h in range(HQ_PER):
            qh = q[:, h * DH:(h + 1) * DH]
            sc = lax.dot_general(qh, kblk[h], (((1,), (1,)), ((), ())),
                                 preferred_element_type=jnp.float32)
            sc = sc * SCALE + addmask
            m = jnp.max(sc, axis=1, keepdims=True)
            w = jnp.exp(sc - m)
            w = (w / jnp.sum(w, axis=1, keepdims=True)).astype(jnp.bfloat16)
            ctxs.append(jnp.dot(w, vblk[h],
                                preferred_element_type=jnp.float32
                                ).astype(jnp.bfloat16))
        part = jnp.dot(jnp.concatenate(ctxs, axis=1), comm_ref[s, 1],
                       preferred_element_type=jnp.float32)
        if s == 0:
            out_ref[...] = part
        else:
            out_ref[...] = out_ref[...] + part


def kernel(x, Wq, K_ext, V_ext, Wo):
    xb = x[0].astype(jnp.bfloat16)
    wq = Wq.astype(jnp.bfloat16)
    wo = Wo.astype(jnp.bfloat16)
    k = jnp.transpose(K_ext[0], (1, 0, 2)).astype(jnp.bfloat16)
    v = jnp.transpose(V_ext[0], (1, 0, 2)).astype(jnp.bfloat16)

    out = pl.pallas_call(
        _body,
        out_shape=jax.ShapeDtypeStruct((SQ, DM), jnp.float32),
        in_specs=[pl.BlockSpec(memory_space=pltpu.VMEM)] * 5,
        out_specs=pl.BlockSpec(memory_space=pltpu.VMEM),
        scratch_shapes=[
            pltpu.VMEM((N_DEV, 2, DM, DM), jnp.bfloat16),
            pltpu.SemaphoreType.DMA((N_DEV,)),
            pltpu.SemaphoreType.DMA((N_DEV,)),
        ],
        compiler_params=pltpu.CompilerParams(collective_id=0),
    )(xb, wq, k, v, wo)
    return out[None]


# baseline (device time: 277850 ns/iter reference)
import jax
import jax.numpy as jnp
from jax import lax
from jax.experimental import pallas as pl
from jax.experimental.pallas import tpu as pltpu

N_DEV = 4
SQ = 1024
SKV = 1024
HQ_PER = 8
DH = 128
DM = 1024
BLK = 64
SCALE = 0.08838834764831843


def _body(x_ref, wq_ref, k_hbm, v_hbm, wo_ref, out_ref,
          comm_ref, kv_ref, send_sems, recv_sems, kv_sems):
    my = lax.axis_index("i")
    left = lax.rem(my + N_DEV - 1, N_DEV)
    right = lax.rem(my + 1, N_DEV)

    barrier = pltpu.get_barrier_semaphore()
    pl.semaphore_signal(barrier, inc=1, device_id=(left,),
                        device_id_type=pl.DeviceIdType.MESH)
    pl.semaphore_signal(barrier, inc=1, device_id=(right,),
                        device_id_type=pl.DeviceIdType.MESH)
    pl.semaphore_wait(barrier, 2)

    comm_ref[0, 0] = wq_ref[...]
    comm_ref[0, 1] = wo_ref[...]

    for h in range(N_DEV - 1):
        rdma = pltpu.make_async_remote_copy(
            src_ref=comm_ref.at[h],
            dst_ref=comm_ref.at[h + 1],
            send_sem=send_sems.at[h],
            recv_sem=recv_sems.at[h + 1],
            device_id=(right,),
            device_id_type=pl.DeviceIdType.MESH,
        )
        rdma.start()
        rdma.wait()

    qrow = lax.broadcasted_iota(jnp.int32, (SQ, 1), 0) + my * SQ
    kcol = lax.broadcasted_iota(jnp.int32, (1, SKV), 1)
    qb = qrow // BLK
    kb = kcol // BLK
    keep = (qb == kb) | (kb == 0) | (lax.rem(qb + kb, 3) == 0)

    xb = x_ref[...]
    for s in range(N_DEV):
        j = lax.rem(my + N_DEV - s, N_DEV)
        kdma = pltpu.make_async_copy(
            k_hbm.at[0, :, pl.ds(j * HQ_PER, HQ_PER), :],
            kv_ref.at[0], kv_sems.at[0])
        vdma = pltpu.make_async_copy(
            v_hbm.at[0, :, pl.ds(j * HQ_PER, HQ_PER), :],
            kv_ref.at[1], kv_sems.at[1])
        kdma.start()
        vdma.start()

        q = jnp.dot(xb, comm_ref[s, 0],
                    preferred_element_type=jnp.float32).astype(jnp.bfloat16)

        kdma.wait()
        vdma.wait()
        kblk = kv_ref[0].astype(jnp.bfloat16)
        vblk = kv_ref[1].astype(jnp.bfloat16)

        ctxs = []
        for h in range(HQ_PER):
            qh = q[:, h * DH:(h + 1) * DH]
            sc = lax.dot_general(qh, kblk[:, h, :], (((1,), (1,)), ((), ())),
                                 preferred_element_type=jnp.float32)
            sc = jnp.where(keep, sc * SCALE, -1e9)
            m = jnp.max(sc, axis=1, keepdims=True)
            w = jnp.exp(sc - m)
            w = (w / jnp.sum(w, axis=1, keepdims=True)).astype(jnp.bfloat16)
            ctxs.append(jnp.dot(w, vblk[:, h, :],
                                preferred_element_type=jnp.float32
                                ).astype(jnp.bfloat16))
        part = jnp.dot(jnp.concatenate(ctxs, axis=1), comm_ref[s, 1],
                       preferred_element_type=jnp.float32)
        if s == 0:
            out_ref[...] = part
        else:
            out_ref[...] = out_ref[...] + part


def kernel(x, Wq, K_ext, V_ext, Wo):
    xb = x[0].astype(jnp.bfloat16)
    wq = Wq.astype(jnp.bfloat16)
    wo = Wo.astype(jnp.bfloat16)

    out = pl.pallas_call(
        _body,
        out_shape=jax.ShapeDtypeStruct((SQ, DM), jnp.float32),
        in_specs=[
            pl.BlockSpec(memory_space=pltpu.VMEM),
            pl.BlockSpec(memory_space=pltpu.VMEM),
            pl.BlockSpec(memory_space=pl.ANY),
            pl.BlockSpec(memory_space=pl.ANY),
            pl.BlockSpec(memory_space=pltpu.VMEM),
        ],
        out_specs=pl.BlockSpec(memory_space=pltpu.VMEM),
        scratch_shapes=[
            pltpu.VMEM((N_DEV, 2, DM, DM), jnp.bfloat16),
            pltpu.VMEM((2, SKV, HQ_PER, DH), jnp.float32),
            pltpu.SemaphoreType.DMA((N_DEV,)),
            pltpu.SemaphoreType.DMA((N_DEV,)),
            pltpu.SemaphoreType.DMA((2,)),
        ],
        compiler_params=pltpu.CompilerParams(
            collective_id=0,
            vmem_limit_bytes=60 * 1024 * 1024,
        ),
    )(xb, wq, K_ext, V_ext, wo)
    return out[None]


# device time: 123016 ns/iter; 2.2586x vs baseline; 2.2586x over previous
import jax
import jax.numpy as jnp
from jax import lax
from jax.experimental import pallas as pl
from jax.experimental.pallas import tpu as pltpu

N_DEV = 4
SQ = 1024
SKV = 1024
HQ_PER = 8
DH = 128
DM = 1024
BLK = 64
SCALE = 0.08838834764831843


def _body(x_ref, wq_ref, k_hbm, v_hbm, wo_ref, out_ref,
          wq_comm, wo_comm, kv_ref, qsend, qrecv, osend, orecv, kv_sems):
    my = lax.axis_index("i")
    left = lax.rem(my + N_DEV - 1, N_DEV)
    right = lax.rem(my + 1, N_DEV)

    barrier = pltpu.get_barrier_semaphore()
    pl.semaphore_signal(barrier, inc=1, device_id=(left,),
                        device_id_type=pl.DeviceIdType.MESH)
    pl.semaphore_signal(barrier, inc=1, device_id=(right,),
                        device_id_type=pl.DeviceIdType.MESH)
    pl.semaphore_wait(barrier, 2)

    wq_comm[0] = wq_ref[...]
    wo_comm[0] = wo_ref[...]

    def ring(comm, h, ssems, rsems, dst):
        return pltpu.make_async_remote_copy(
            src_ref=comm.at[h], dst_ref=comm.at[h + 1],
            send_sem=ssems.at[h], recv_sem=rsems.at[h + 1],
            device_id=(dst,), device_id_type=pl.DeviceIdType.MESH)

    qhops = [ring(wq_comm, h, qsend, qrecv, right) for h in range(N_DEV - 1)]
    ohops = [ring(wo_comm, h, osend, orecv, left) for h in range(N_DEV - 1)]

    def kv_fetch(j):
        kd = pltpu.make_async_copy(
            k_hbm.at[0, :, pl.ds(j * HQ_PER, HQ_PER), :],
            kv_ref.at[0], kv_sems.at[0])
        vd = pltpu.make_async_copy(
            v_hbm.at[0, :, pl.ds(j * HQ_PER, HQ_PER), :],
            kv_ref.at[1], kv_sems.at[1])
        kd.start()
        vd.start()
        return (kd, vd)

    qrow = lax.broadcasted_iota(jnp.int32, (SQ, 1), 0) + my * SQ
    kcol = lax.broadcasted_iota(jnp.int32, (1, SKV), 1)
    qb = qrow // BLK
    kb = kcol // BLK
    keep = (qb == kb) | (kb == 0) | (lax.rem(qb + kb, 3) == 0)

    xb = x_ref[...]

    def attn(j, wq_slot):
        descs = kv_fetch(j)
        q = jnp.dot(xb, wq_comm[wq_slot],
                    preferred_element_type=jnp.float32).astype(jnp.bfloat16)
        for d in descs:
            d.wait()
        kblk = kv_ref[0].astype(jnp.bfloat16)
        vblk = kv_ref[1].astype(jnp.bfloat16)
        ctxs = []
        for h in range(HQ_PER):
            qh = q[:, h * DH:(h + 1) * DH]
            sc = lax.dot_general(qh, kblk[:, h, :], (((1,), (1,)), ((), ())),
                                 preferred_element_type=jnp.float32)
            w = jnp.where(keep, jnp.exp(sc * SCALE), 0.0)
            wsum = jnp.sum(w, axis=1, keepdims=True)
            ctx = jnp.dot(w.astype(jnp.bfloat16), vblk[:, h, :],
                          preferred_element_type=jnp.float32)
            ctxs.append((ctx / wsum).astype(jnp.bfloat16))
        return jnp.concatenate(ctxs, axis=1)

    def proj(ctx, wo_slot):
        return jnp.dot(ctx, wo_comm[wo_slot],
                       preferred_element_type=jnp.float32)

    qhops[0].start()
    ohops[0].start()
    out_ref[...] = proj(attn(my, 0), 0)

    qhops[0].wait_recv()
    ohops[0].wait_recv()
    qhops[1].start()
    ohops[1].start()
    ctx1 = attn(lax.rem(my + 3, N_DEV), 1)

    qhops[1].wait_recv()
    ohops[1].wait_recv()
    qhops[2].start()
    ohops[2].start()
    out_ref[...] = out_ref[...] + proj(attn(lax.rem(my + 2, N_DEV), 2), 2)

    qhops[2].wait_recv()
    ohops[2].wait_recv()
    out_ref[...] = out_ref[...] + proj(attn(lax.rem(my + 1, N_DEV), 3), 1)
    out_ref[...] = out_ref[...] + proj(ctx1, 3)

    for hop in qhops + ohops:
        hop.wait_send()


def kernel(x, Wq, K_ext, V_ext, Wo):
    xb = x[0].astype(jnp.bfloat16)
    wq = Wq.astype(jnp.bfloat16)
    wo = Wo.astype(jnp.bfloat16)

    out = pl.pallas_call(
        _body,
        out_shape=jax.ShapeDtypeStruct((SQ, DM), jnp.float32),
        in_specs=[
            pl.BlockSpec(memory_space=pltpu.MemorySpace.VMEM),
            pl.BlockSpec(memory_space=pltpu.MemorySpace.VMEM),
            pl.BlockSpec(memory_space=pl.ANY),
            pl.BlockSpec(memory_space=pl.ANY),
            pl.BlockSpec(memory_space=pltpu.MemorySpace.VMEM),
        ],
        out_specs=pl.BlockSpec(memory_space=pltpu.MemorySpace.VMEM),
        scratch_shapes=[
            pltpu.VMEM((N_DEV, DM, DM), jnp.bfloat16),
            pltpu.VMEM((N_DEV, DM, DM), jnp.bfloat16),
            pltpu.VMEM((2, SKV, HQ_PER, DH), jnp.float32),
            pltpu.SemaphoreType.DMA((N_DEV,)),
            pltpu.SemaphoreType.DMA((N_DEV,)),
            pltpu.SemaphoreType.DMA((N_DEV,)),
            pltpu.SemaphoreType.DMA((N_DEV,)),
            pltpu.SemaphoreType.DMA((2,)),
        ],
        compiler_params=pltpu.CompilerParams(
            collective_id=0,
            vmem_limit_bytes=63 * 1024 * 1024,
        ),
    )(xb, wq, K_ext, V_ext, wo)
    return out[None]


# device time: 118533 ns/iter; 2.3441x vs baseline; 1.0378x over previous
import jax
import jax.numpy as jnp
from jax import lax
from jax.experimental import pallas as pl
from jax.experimental.pallas import tpu as pltpu

N_DEV = 4
SQ = 1024
SKV = 1024
HQ_PER = 8
DH = 128
DM = 1024
BLK = 64
SCALE = 0.08838834764831843


def _body(x_ref, wq_ref, k_hbm, v_hbm, wo_ref, out_ref,
          wq_comm, wo_comm, kv_ref, qsend, qrecv, osend, orecv, kv_sems):
    my = lax.axis_index("i")
    left = lax.rem(my + N_DEV - 1, N_DEV)
    right = lax.rem(my + 1, N_DEV)

    barrier = pltpu.get_barrier_semaphore()
    pl.semaphore_signal(barrier, inc=1, device_id=(left,),
                        device_id_type=pl.DeviceIdType.MESH)
    pl.semaphore_signal(barrier, inc=1, device_id=(right,),
                        device_id_type=pl.DeviceIdType.MESH)
    pl.semaphore_wait(barrier, 2)

    wq_comm[0] = wq_ref[...]
    wo_comm[0] = wo_ref[...]

    def ring(comm, h, ssems, rsems, dst):
        return pltpu.make_async_remote_copy(
            src_ref=comm.at[h], dst_ref=comm.at[h + 1],
            send_sem=ssems.at[h], recv_sem=rsems.at[h + 1],
            device_id=(dst,), device_id_type=pl.DeviceIdType.MESH)

    qhops = [ring(wq_comm, h, qsend, qrecv, right) for h in range(N_DEV - 1)]
    ohops = [ring(wo_comm, h, osend, orecv, left) for h in range(N_DEV - 1)]

    def kv_fetch(j):
        kd = pltpu.make_async_copy(
            k_hbm.at[0, :, pl.ds(j * HQ_PER, HQ_PER), :],
            kv_ref.at[0], kv_sems.at[0])
        vd = pltpu.make_async_copy(
            v_hbm.at[0, :, pl.ds(j * HQ_PER, HQ_PER), :],
            kv_ref.at[1], kv_sems.at[1])
        kd.start()
        vd.start()
        return (kd, vd)

    qrow = lax.broadcasted_iota(jnp.int32, (SQ, 1), 0) + my * SQ
    kcol = lax.broadcasted_iota(jnp.int32, (1, SKV), 1)
    qb = qrow // BLK
    kb = kcol // BLK
    keep = (qb == kb) | (kb == 0) | (lax.rem(qb + kb, 3) == 0)

    xb = (x_ref[...] * SCALE).astype(jnp.bfloat16)

    def attn(j, wq_slot):
        descs = kv_fetch(j)
        q = jnp.dot(xb, wq_comm[wq_slot],
                    preferred_element_type=jnp.float32).astype(jnp.bfloat16)
        for d in descs:
            d.wait()
        kblk = kv_ref[0].astype(jnp.bfloat16)
        vblk = kv_ref[1].astype(jnp.bfloat16)
        ctxs = []
        for h in range(HQ_PER):
            qh = q[:, h * DH:(h + 1) * DH]
            sc = lax.dot_general(qh, kblk[:, h, :], (((1,), (1,)), ((), ())),
                                 preferred_element_type=jnp.float32)
            w = jnp.where(keep, jnp.exp(sc.astype(jnp.bfloat16)),
                          jnp.bfloat16(0))
            wsum = jnp.sum(w, axis=1, keepdims=True, dtype=jnp.float32)
            ctx = jnp.dot(w, vblk[:, h, :],
                          preferred_element_type=jnp.float32)
            ctxs.append((ctx / wsum).astype(jnp.bfloat16))
        return jnp.concatenate(ctxs, axis=1)

    def proj(ctx, wo_slot):
        return jnp.dot(ctx, wo_comm[wo_slot],
                       preferred_element_type=jnp.float32)

    qhops[0].start()
    ohops[0].start()
    out_ref[...] = proj(attn(my, 0), 0)

    qhops[0].wait_recv()
    ohops[0].wait_recv()
    qhops[1].start()
    ohops[1].start()
    ctx1 = attn(lax.rem(my + 3, N_DEV), 1)

    qhops[1].wait_recv()
    ohops[1].wait_recv()
    qhops[2].start()
    ohops[2].start()
    out_ref[...] = out_ref[...] + proj(attn(lax.rem(my + 2, N_DEV), 2), 2)

    qhops[2].wait_recv()
    ohops[2].wait_recv()
    out_ref[...] = out_ref[...] + proj(attn(lax.rem(my + 1, N_DEV), 3), 1)
    out_ref[...] = out_ref[...] + proj(ctx1, 3)

    for hop in qhops + ohops:
        hop.wait_send()


def kernel(x, Wq, K_ext, V_ext, Wo):
    xb = x[0].astype(jnp.bfloat16)
    wq = Wq.astype(jnp.bfloat16)
    wo = Wo.astype(jnp.bfloat16)

    out = pl.pallas_call(
        _body,
        out_shape=jax.ShapeDtypeStruct((SQ, DM), jnp.float32),
        in_specs=[
            pl.BlockSpec(memory_space=pltpu.MemorySpace.VMEM),
            pl.BlockSpec(memory_space=pltpu.MemorySpace.VMEM),
            pl.BlockSpec(memory_space=pl.ANY),
            pl.BlockSpec(memory_space=pl.ANY),
            pl.BlockSpec(memory_space=pltpu.MemorySpace.VMEM),
        ],
        out_specs=pl.BlockSpec(memory_space=pltpu.MemorySpace.VMEM),
        scratch_shapes=[
            pltpu.VMEM((N_DEV, DM, DM), jnp.bfloat16),
            pltpu.VMEM((N_DEV, DM, DM), jnp.bfloat16),
            pltpu.VMEM((2, SKV, HQ_PER, DH), jnp.float32),
            pltpu.SemaphoreType.DMA((N_DEV,)),
            pltpu.SemaphoreType.DMA((N_DEV,)),
            pltpu.SemaphoreType.DMA((N_DEV,)),
            pltpu.SemaphoreType.DMA((N_DEV,)),
            pltpu.SemaphoreType.DMA((2,)),
        ],
        compiler_params=pltpu.CompilerParams(
            collective_id=0,
            vmem_limit_bytes=63 * 1024 * 1024,
        ),
    )(xb, wq, K_ext, V_ext, wo)
    return out[None]


# device time: 116640 ns/iter; 2.3821x vs baseline; 1.0162x over previous
import jax
import jax.numpy as jnp
from jax import lax
from jax.experimental import pallas as pl
from jax.experimental.pallas import tpu as pltpu

N_DEV = 4
SQ = 1024
SKV = 1024
HQ_PER = 8
DH = 128
DM = 1024
BLK = 64
SCALE = 0.08838834764831843


def _body(x_ref, wq_ref, k_hbm, v_hbm, wo_ref, out_ref,
          wq_comm, wo_comm, kv_ref, qsend, qrecv, osend, orecv,
          q2send, q2recv, kv_sems):
    my = lax.axis_index("i")
    left = lax.rem(my + N_DEV - 1, N_DEV)
    right = lax.rem(my + 1, N_DEV)

    barrier = pltpu.get_barrier_semaphore()
    pl.semaphore_signal(barrier, inc=1, device_id=(left,),
                        device_id_type=pl.DeviceIdType.MESH)
    pl.semaphore_signal(barrier, inc=1, device_id=(right,),
                        device_id_type=pl.DeviceIdType.MESH)
    pl.semaphore_wait(barrier, 2)

    wq_comm[0] = wq_ref[...]
    wo_comm[0] = wo_ref[...]

    def ring(comm, h, ssems, rsems, dst):
        return pltpu.make_async_remote_copy(
            src_ref=comm.at[h], dst_ref=comm.at[h + 1],
            send_sem=ssems.at[h], recv_sem=rsems.at[h + 1],
            device_id=(dst,), device_id_type=pl.DeviceIdType.MESH)

    qhops = [ring(wq_comm, h, qsend, qrecv, right) for h in range(N_DEV - 2)]
    ohops = [ring(wo_comm, h, osend, orecv, left) for h in range(N_DEV - 1)]
    qh2 = [
        pltpu.make_async_remote_copy(
            src_ref=wq_comm.at[2, :, pl.ds(p * (DM // 2), DM // 2)],
            dst_ref=wq_comm.at[3, :, pl.ds(p * (DM // 2), DM // 2)],
            send_sem=q2send.at[p], recv_sem=q2recv.at[p],
            device_id=(right,), device_id_type=pl.DeviceIdType.MESH)
        for p in range(2)
    ]

    def kv_fetch(j):
        kd = pltpu.make_async_copy(
            k_hbm.at[0, :, pl.ds(j * HQ_PER, HQ_PER), :],
            kv_ref.at[0], kv_sems.at[0])
        vd = pltpu.make_async_copy(
            v_hbm.at[0, :, pl.ds(j * HQ_PER, HQ_PER), :],
            kv_ref.at[1], kv_sems.at[1])
        kd.start()
        vd.start()
        return (kd, vd)

    qrow = lax.broadcasted_iota(jnp.int32, (SQ, 1), 0) + my * SQ
    kcol = lax.broadcasted_iota(jnp.int32, (1, SKV), 1)
    qb = qrow // BLK
    kb = kcol // BLK
    keep = (qb == kb) | (kb == 0) | (lax.rem(qb + kb, 3) == 0)

    xb = (x_ref[...] * SCALE).astype(jnp.bfloat16)

    def heads_ctx(q, kblk, vblk, h0):
        ctxs = []
        for h in range(q.shape[1] // DH):
            qh = q[:, h * DH:(h + 1) * DH]
            sc = lax.dot_general(qh, kblk[:, h0 + h, :],
                                 (((1,), (1,)), ((), ())),
                                 preferred_element_type=jnp.float32)
            w = jnp.where(keep, jnp.exp(sc.astype(jnp.bfloat16)),
                          jnp.bfloat16(0))
            wsum = jnp.sum(w, axis=1, keepdims=True, dtype=jnp.float32)
            ctx = jnp.dot(w, vblk[:, h0 + h, :],
                          preferred_element_type=jnp.float32)
            ctxs.append((ctx / wsum).astype(jnp.bfloat16))
        return jnp.concatenate(ctxs, axis=1)

    def attn(j, wq_slot):
        descs = kv_fetch(j)
        q = jnp.dot(xb, wq_comm[wq_slot],
                    preferred_element_type=jnp.float32).astype(jnp.bfloat16)
        for d in descs:
            d.wait()
        kblk = kv_ref[0].astype(jnp.bfloat16)
        vblk = kv_ref[1].astype(jnp.bfloat16)
        return heads_ctx(q, kblk, vblk, 0)

    def proj(ctx, wo_slot):
        return jnp.dot(ctx, wo_comm[wo_slot],
                       preferred_element_type=jnp.float32)

    qhops[0].start()
    ohops[0].start()
    out_ref[...] = proj(attn(my, 0), 0)

    qhops[0].wait_recv()
    ohops[0].wait_recv()
    qhops[1].start()
    ohops[1].start()
    ctx1 = attn(lax.rem(my + 3, N_DEV), 1)

    qhops[1].wait_recv()
    ohops[1].wait_recv()
    qh2[0].start()
    qh2[1].start()
    ohops[2].start()
    out_ref[...] = out_ref[...] + proj(attn(lax.rem(my + 2, N_DEV), 2), 2)

    kd, vd = kv_fetch(lax.rem(my + 1, N_DEV))
    kd.wait()
    vd.wait()
    kblk3 = kv_ref[0].astype(jnp.bfloat16)
    vblk3 = kv_ref[1].astype(jnp.bfloat16)
    half = DM // 2
    qh2[0].wait_recv()
    q3a = jnp.dot(xb, wq_comm[3, :, :half],
                  preferred_element_type=jnp.float32).astype(jnp.bfloat16)
    ctx3a = heads_ctx(q3a, kblk3, vblk3, 0)
    out_ref[...] = out_ref[...] + jnp.dot(
        ctx3a, wo_comm[1, :half, :], preferred_element_type=jnp.float32)
    qh2[1].wait_recv()
    q3b = jnp.dot(xb, wq_comm[3, :, half:],
                  preferred_element_type=jnp.float32).astype(jnp.bfloat16)
    ctx3b = heads_ctx(q3b, kblk3, vblk3, HQ_PER // 2)
    out_ref[...] = out_ref[...] + jnp.dot(
        ctx3b, wo_comm[1, half:, :], preferred_element_type=jnp.float32)

    ohops[2].wait_recv()
    out_ref[...] = out_ref[...] + proj(ctx1, 3)

    for hop in qhops + ohops + qh2:
        hop.wait_send()


def kernel(x, Wq, K_ext, V_ext, Wo):
    xb = x[0].astype(jnp.bfloat16)
    wq = Wq.astype(jnp.bfloat16)
    wo = Wo.astype(jnp.bfloat16)

    out = pl.pallas_call(
        _body,
        out_shape=jax.ShapeDtypeStruct((SQ, DM), jnp.float32),
        in_specs=[
            pl.BlockSpec(memory_space=pltpu.MemorySpace.VMEM),
            pl.BlockSpec(memory_space=pltpu.MemorySpace.VMEM),
            pl.BlockSpec(memory_space=pl.ANY),
            pl.BlockSpec(memory_space=pl.ANY),
            pl.BlockSpec(memory_space=pltpu.MemorySpace.VMEM),
        ],
        out_specs=pl.BlockSpec(memory_space=pltpu.MemorySpace.VMEM),
        scratch_shapes=[
            pltpu.VMEM((N_DEV, DM, DM), jnp.bfloat16),
            pltpu.VMEM((N_DEV, DM, DM), jnp.bfloat16),
            pltpu.VMEM((2, SKV, HQ_PER, DH), jnp.float32),
            pltpu.SemaphoreType.DMA((N_DEV,)),
            pltpu.SemaphoreType.DMA((N_DEV,)),
            pltpu.SemaphoreType.DMA((N_DEV,)),
            pltpu.SemaphoreType.DMA((N_DEV,)),
            pltpu.SemaphoreType.DMA((2,)),
            pltpu.SemaphoreType.DMA((2,)),
            pltpu.SemaphoreType.DMA((2,)),
        ],
        compiler_params=pltpu.CompilerParams(
            collective_id=0,
            vmem_limit_bytes=63 * 1024 * 1024,
        ),
    )(xb, wq, K_ext, V_ext, wo)
    return out[None]
